# baseline (speedup 1.0000x reference)
import functools

import jax
import jax.numpy as jnp
from jax import lax
from jax.experimental import pallas as pl
from jax.experimental.pallas import tpu as pltpu
from jax.experimental.pallas import tpu_sc as plsc

VOCAB = 1000
VPAD = 1024
EMBED_DIM = 128
NTOK = 4096 * 20
NW = 32
ROWS_PER_W = NTOK // NW
CHUNK = 40
NCHUNK = ROWS_PER_W // CHUNK


def _proj_table_kernel(emb_ref, w_ref, b_ref, p_ref):
    p_ref[...] = lax.dot_general(
        emb_ref[...], w_ref[...],
        (((1,), (1,)), ((), ())),
        preferred_element_type=jnp.float32,
    ) + b_ref[...]


_mesh = plsc.VectorSubcoreMesh(
    core_axis_name="c", subcore_axis_name="s", num_cores=2, num_subcores=16
)


@functools.partial(
    pl.kernel,
    out_type=jax.ShapeDtypeStruct((NTOK, VPAD), jnp.float32),
    mesh=_mesh,
    scratch_types=[
        pltpu.VMEM((ROWS_PER_W,), jnp.int32),
        pltpu.VMEM((CHUNK, VPAD), jnp.float32),
        pltpu.VMEM((CHUNK, VPAD), jnp.float32),
        pltpu.SemaphoreType.DMA,
        pltpu.SemaphoreType.DMA,
        pltpu.SemaphoreType.DMA,
        pltpu.SemaphoreType.DMA,
    ],
)
def _gather_rows(table_hbm, idx_hbm, out_hbm, idx_v, rows0, rows1,
                 sg0, sg1, sw0, sw1):
    wid = lax.axis_index("s") * 2 + lax.axis_index("c")
    w_base = wid * ROWS_PER_W
    rows = (rows0, rows1)
    sg = (sg0, sg1)
    sw = (sw0, sw1)

    pltpu.sync_copy(idx_hbm.at[pl.ds(w_base, ROWS_PER_W)], idx_v)

    def start_gather(i, b):
        pltpu.async_copy(table_hbm.at[idx_v.at[pl.ds(i * CHUNK, CHUNK)]],
                         rows[b], sg[b])

    def wait_gather(i, b):
        pltpu.make_async_copy(table_hbm.at[idx_v.at[pl.ds(i * CHUNK, CHUNK)]],
                              rows[b], sg[b]).wait()

    def start_write(i, b):
        pltpu.async_copy(rows[b], out_hbm.at[pl.ds(w_base + i * CHUNK, CHUNK)],
                         sw[b])

    def wait_write(i, b):
        pltpu.make_async_copy(rows[b],
                              out_hbm.at[pl.ds(w_base + i * CHUNK, CHUNK)],
                              sw[b]).wait()

    start_gather(0, 0)
    start_gather(1, 1)
    wait_gather(0, 0)
    start_write(0, 0)
    wait_write(0, 0)
    start_gather(2, 0)
    wait_gather(1, 1)
    start_write(1, 1)

    def body(j, carry):
        i0 = 2 * j
        wait_write(i0 - 1, 1)
        start_gather(i0 + 1, 1)
        wait_gather(i0, 0)
        start_write(i0, 0)
        i1 = i0 + 1
        wait_write(i1 - 1, 0)
        start_gather(i1 + 1, 0)
        wait_gather(i1, 1)
        start_write(i1, 1)
        return carry

    lax.fori_loop(1, NCHUNK // 2 - 1, body, 0)

    i0 = NCHUNK - 2
    wait_write(i0 - 1, 1)
    start_gather(i0 + 1, 1)
    wait_gather(i0, 0)
    start_write(i0, 0)
    wait_gather(i0 + 1, 1)
    start_write(i0 + 1, 1)
    wait_write(i0, 0)
    wait_write(i0 + 1, 1)


def kernel(x, emb_table, W, b):
    w_pad = jnp.zeros((VPAD, EMBED_DIM), jnp.float32).at[:VOCAB].set(W)
    b_pad = jnp.zeros((1, VPAD), jnp.float32).at[0, :VOCAB].set(b)
    P = pl.pallas_call(
        _proj_table_kernel,
        out_shape=jax.ShapeDtypeStruct((VOCAB, VPAD), jnp.float32),
    )(emb_table, w_pad, b_pad)
    idx = x.reshape(-1).astype(jnp.int32)
    logits = _gather_rows(P, idx)
    return logits[:, :VOCAB].reshape(x.shape[0], x.shape[1], VOCAB)


# speedup vs baseline: 1.2638x; 1.2638x over previous
import functools

import jax
import jax.numpy as jnp
from jax import lax
from jax.experimental import pallas as pl
from jax.experimental.pallas import tpu as pltpu
from jax.experimental.pallas import tpu_sc as plsc

VOCAB = 1000
VPAD = 1024
EMBED_DIM = 128
BATCH = 4096
SEQ = 20
SEQ_PAD = 24
NLT = VPAD // 128
NG = 9
TAIL_A = 896
TAIL_B = 888
NW = 32
BATCH_PER_W = BATCH // NW


def _proj_table_kernel(emb_ref, w_ref, wta_ref, wtb_ref, b_ref, bta_ref,
                       btb_ref, p_ref, pta_ref, ptb_ref):
    h = emb_ref[...]

    def nt(w):
        return lax.dot_general(h, w, (((1,), (1,)), ((), ())),
                               preferred_element_type=jnp.float32)

    p_ref[...] = nt(w_ref[...]) + b_ref[...]
    pta_ref[...] = nt(wta_ref[...]) + bta_ref[...]
    ptb_ref[...] = nt(wtb_ref[...]) + btb_ref[...]


_mesh = plsc.VectorSubcoreMesh(
    core_axis_name="c", subcore_axis_name="s", num_cores=2, num_subcores=16
)


@functools.partial(
    pl.kernel,
    out_type=jax.ShapeDtypeStruct((BATCH, SEQ, VOCAB), jnp.float32),
    mesh=_mesh,
    scratch_types=[
        pltpu.VMEM((BATCH_PER_W * NG * SEQ_PAD,), jnp.int32),
        pltpu.VMEM((SEQ, VOCAB), jnp.float32),
        pltpu.VMEM((SEQ, VOCAB), jnp.float32),
        pltpu.VMEM((SEQ, 128), jnp.float32),
        pltpu.VMEM((SEQ, 128), jnp.float32),
        pltpu.VMEM((SEQ, 128), jnp.float32),
        pltpu.VMEM((SEQ, 128), jnp.float32),
        pltpu.SemaphoreType.DMA,
        pltpu.SemaphoreType.DMA,
        pltpu.SemaphoreType.DMA,
        pltpu.SemaphoreType.DMA,
    ],
)
def _gather_rows(table_hbm, ta_hbm, tb_hbm, idx_hbm, out_hbm, idx_v,
                 rows0, rows1, ta0, ta1, tb0, tb1, sg0, sg1, sw0, sw1):
    wid = lax.axis_index("s") * 2 + lax.axis_index("c")
    w_base = wid * BATCH_PER_W
    rows = (rows0, rows1)
    tas = (ta0, ta1)
    tbs = (tb0, tb1)
    sg = (sg0, sg1)
    sw = (sw0, sw1)

    pltpu.sync_copy(
        idx_hbm.at[pl.ds(w_base * NG * SEQ_PAD, BATCH_PER_W * NG * SEQ_PAD)],
        idx_v)

    def gather_parts(i, b):
        def ilist(g):
            return idx_v.at[pl.ds((i * NG + g) * SEQ_PAD, SEQ)]
        for c in range(NLT - 1):
            yield (table_hbm.at[ilist(c)],
                   rows[b].at[:, pl.ds(c * 128, 128)], sg[b])
        yield (ta_hbm.at[ilist(7)], tas[b], sg[b])
        yield (tb_hbm.at[ilist(8)], tbs[b], sg[b])

    def start_gather(i, b):
        for src, dst, sem in gather_parts(i, b):
            pltpu.async_copy(src, dst, sem)

    def wait_gather(i, b):
        for src, dst, sem in gather_parts(i, b):
            pltpu.make_async_copy(src, dst, sem).wait()

    def fill_tail(b):
        for r in range(SEQ):
            rows[b][r, pl.ds(984, 16)] = tbs[b][r, pl.ds(96, 16)]
            for k in range(6):
                rows[b][r, pl.ds(TAIL_A + 16 * k, 16)] = (
                    tas[b][r, pl.ds(16 * k, 16)])

    def start_write(i, b):
        pltpu.async_copy(rows[b], out_hbm.at[w_base + i], sw[b])

    def wait_write(i, b):
        pltpu.make_async_copy(rows[b], out_hbm.at[w_base + i], sw[b]).wait()

    start_gather(0, 0)
    start_gather(1, 1)
    wait_gather(0, 0)
    fill_tail(0)
    start_write(0, 0)
    wait_write(0, 0)
    start_gather(2, 0)
    wait_gather(1, 1)
    fill_tail(1)
    start_write(1, 1)

    def body(j, carry):
        i0 = 2 * j
        wait_write(i0 - 1, 1)
        start_gather(i0 + 1, 1)
        wait_gather(i0, 0)
        fill_tail(0)
        start_write(i0, 0)
        i1 = i0 + 1
        wait_write(i1 - 1, 0)
        start_gather(i1 + 1, 0)
        wait_gather(i1, 1)
        fill_tail(1)
        start_write(i1, 1)
        return carry

    lax.fori_loop(1, BATCH_PER_W // 2 - 1, body, 0)

    i0 = BATCH_PER_W - 2
    wait_write(i0 - 1, 1)
    start_gather(i0 + 1, 1)
    wait_gather(i0, 0)
    fill_tail(0)
    start_write(i0, 0)
    wait_gather(i0 + 1, 1)
    fill_tail(1)
    start_write(i0 + 1, 1)
    wait_write(i0, 0)
    wait_write(i0 + 1, 1)


def kernel(x, emb_table, W, b):
    w_pad = jnp.zeros((VPAD, EMBED_DIM), jnp.float32).at[:VOCAB].set(W)
    b_pad = jnp.zeros((1, VPAD), jnp.float32).at[0, :VOCAB].set(b)
    P, P_ta, P_tb = pl.pallas_call(
        _proj_table_kernel,
        out_shape=(jax.ShapeDtypeStruct((VOCAB, VPAD), jnp.float32),
                   jax.ShapeDtypeStruct((VOCAB, 128), jnp.float32),
                   jax.ShapeDtypeStruct((VOCAB, 128), jnp.float32)),
    )(emb_table, w_pad, w_pad[TAIL_A:TAIL_A + 128],
      w_pad[TAIL_B:TAIL_B + 128], b_pad, b_pad[:, TAIL_A:TAIL_A + 128],
      b_pad[:, TAIL_B:TAIL_B + 128])
    p_sub = P.reshape(VOCAB * NLT, 128)
    xb = x.astype(jnp.int32)
    xp = jnp.pad(xb, ((0, 0), (0, SEQ_PAD - SEQ)))
    gvec = jnp.arange(NG, dtype=jnp.int32)[None, :, None]
    idx_ord = jnp.where(gvec < NLT - 1, 8 * xp[:, None, :] + gvec,
                        xp[:, None, :]).reshape(-1)
    return _gather_rows(p_sub, P_ta, P_tb, idx_ord)
